# baseline (speedup 1.0000x reference)
import functools

import jax
import jax.numpy as jnp
import numpy as np
from jax import lax
from jax.experimental import pallas as pl
from jax.experimental.pallas import tpu as pltpu
from jax.experimental.pallas import tpu_sc as plsc

B, D, T, S, NG, P = 16, 2, 8, 64, 6, 8
NS = B * (D * T * S + S + P * S)
NGN = B * (D * T * NG + NG + P * NG)
N = NS + NGN
E = 262144
EMBD = 128
GSTATE = 64
NFREQ = 10
DPOS = 3 * (1 + 2 * NFREQ)

BLK = 1024
N_PAD = 28672
NBLK = N_PAD // BLK
NSBLK = NS // BLK
G_PAD = N_PAD - NS

NCORE, NSUB = 2, 16
N_SC = 28032
RPT = N_SC // NSUB
CH = 128
EROWS = E // CH
ER_PER_TILE = EROWS // NSUB
BATCH_R = 8
NBATCH = ER_PER_TILE // BATCH_R
NRING = 2
DEG_HI = 224


def _np_pos_enc(x):
    freqs = 2.0 ** np.arange(NFREQ)
    xf = x[:, :, None] * freqs
    enc = np.concatenate([np.sin(xf), np.cos(xf)], axis=-1)
    enc = enc.reshape(x.shape[0], 3 * 2 * NFREQ)
    return np.concatenate([x, enc], axis=-1).astype(np.float32)


def _static_gripper():
    node1 = np.broadcast_to(np.arange(NG)[None, None, None, :], (B, D, T, NG)).reshape(-1)
    node2 = np.broadcast_to(np.arange(NG)[None, :], (B, NG)).reshape(-1)
    node3 = np.broadcast_to(np.arange(NG)[None, None, :], (B, P, NG)).reshape(-1)
    t_act = np.broadcast_to(np.arange(P)[None, :, None], (B, P, NG)).reshape(-1)
    gripper_node = np.concatenate([node1, node2, node3])
    embd_idx = np.concatenate([node1, node2, node3 + NG * t_act])
    gnp_pos = np.array([[0.0, 0.0, 0.0], [0.0, 0.0, -0.03], [0.0, 0.03, 0.0],
                        [0.0, -0.03, 0.0], [0.0, 0.03, 0.03], [0.0, -0.03, 0.03]],
                       dtype=np.float32) * 2.0
    pe_g = _np_pos_enc(gnp_pos[gripper_node])
    pe_g_pad = np.zeros((G_PAD, 64), np.float32)
    pe_g_pad[:NGN, :DPOS] = pe_g
    onehot = np.zeros((G_PAD, 64), np.float32)
    onehot[np.arange(NGN), embd_idx] = 1.0
    return pe_g_pad, onehot


_PE_G_PAD, _ONEHOT_PAD = _static_gripper()
_FREQS = (2.0 ** np.arange(NFREQ, dtype=np.float32)).reshape(1, NFREQ)


def _pe_scene(pos):
    colb = lax.broadcasted_iota(jnp.int32, (BLK, 64), 1)
    cm3 = colb - 3
    istrig = (colb >= 3) & (colb < 63)
    iscos = istrig & (lax.rem(cm3, 20) >= 10)
    coord = jnp.clip(jnp.where(colb < 3, colb, cm3 // 20), 0, 2)
    k10 = lax.rem(lax.rem(cm3 + 20, 20), 10)
    freq = jnp.exp2(k10.astype(jnp.float32))
    val = jnp.where(coord == 0, pos[:, 0:1],
                    jnp.where(coord == 1, pos[:, 1:2], pos[:, 2:3]))
    ang = val * jnp.where(istrig, freq, jnp.float32(1.0))
    shifted = ang + jnp.where(iscos, jnp.float32(np.pi / 2), jnp.float32(0))
    pe = jnp.where(istrig, jnp.sin(shifted), ang)
    return jnp.where(colb >= 63, jnp.float32(0.0), pe)


def _gemb_gst(oh_ref, tbl_ref, gw_ref, wg_ref, bg_ref):
    gemb = jnp.dot(oh_ref[...], tbl_ref[...], preferred_element_type=jnp.float32)
    gst = gw_ref[...] * wg_ref[0:1, :] + bg_ref[0:1, :]
    return jnp.concatenate([gemb, gst], axis=1)


def _featA_body(pos_ref, gw_ref, peg_ref, oh_ref, wsum_ref, wes_ref,
                wg_ref, bg_ref, tbl_ref, cc_ref):
    k = pl.program_id(0)

    @pl.when(k < NSBLK)
    def _scene():
        pe = _pe_scene(pos_ref[...])
        cc_ref[...] = jnp.dot(pe, wsum_ref[...], preferred_element_type=jnp.float32)

    @pl.when(k >= NSBLK)
    def _grip():
        x = _gemb_gst(oh_ref, tbl_ref, gw_ref, wg_ref, bg_ref)
        a = jnp.dot(peg_ref[...], wes_ref[...], preferred_element_type=jnp.float32)
        cc_ref[...] = x + a


def _featB_body(pos_ref, gw_ref, peg_ref, oh_ref, ws_ref, wed_ref,
                wg_ref, bg_ref, tbl_ref, xf_ref, bd_ref):
    k = pl.program_id(0)

    @pl.when(k < NSBLK)
    def _scene():
        pe = _pe_scene(pos_ref[...])
        xf_ref[...] = jnp.dot(pe, ws_ref[...], preferred_element_type=jnp.float32)
        bd_ref[...] = jnp.dot(pe, wed_ref[...], preferred_element_type=jnp.float32)

    @pl.when(k >= NSBLK)
    def _grip():
        peg = peg_ref[...]
        xf_ref[...] = _gemb_gst(oh_ref, tbl_ref, gw_ref, wg_ref, bg_ref)
        bd_ref[...] = jnp.dot(peg, wed_ref[...], preferred_element_type=jnp.float32)


_SCENE_IX = lambda k: (jnp.minimum(k, NSBLK - 1), 0)
_GRIP_IX = lambda k: (jnp.maximum(k - NSBLK, 0), 0)
_FULL_IX = lambda k: (0, 0)
_COMMON_SPECS = [
    pl.BlockSpec((BLK, 3), _SCENE_IX),
    pl.BlockSpec((BLK, 1), _GRIP_IX),
    pl.BlockSpec((BLK, 64), _GRIP_IX),
    pl.BlockSpec((BLK, 64), _GRIP_IX),
]
_W_SPECS = [
    pl.BlockSpec((8, 64), _FULL_IX),
    pl.BlockSpec((8, 64), _FULL_IX),
    pl.BlockSpec((64, 64), _FULL_IX),
]


def _featA(scene_pos, gw_pad, wsum_pad, wes_pad, wg_pad, bg_pad, tbl_pad):
    f32 = jnp.float32
    return pl.pallas_call(
        _featA_body,
        grid=(NBLK,),
        in_specs=(_COMMON_SPECS
                  + [pl.BlockSpec((64, 128), _FULL_IX),
                     pl.BlockSpec((64, 128), _FULL_IX)] + _W_SPECS),
        out_specs=pl.BlockSpec((BLK, 128), lambda k: (k, 0)),
        out_shape=jax.ShapeDtypeStruct((N_PAD, 128), f32),
    )(scene_pos, gw_pad, _PE_G_PAD, _ONEHOT_PAD, wsum_pad, wes_pad,
      wg_pad, bg_pad, tbl_pad)


def _featB(scene_pos, gw_pad, ws_pad, wed_pad, wg_pad, bg_pad, tbl_pad):
    f32 = jnp.float32
    return pl.pallas_call(
        _featB_body,
        grid=(NBLK,),
        in_specs=(_COMMON_SPECS
                  + [pl.BlockSpec((64, 128), _FULL_IX),
                     pl.BlockSpec((64, 128), _FULL_IX)] + _W_SPECS),
        out_specs=[
            pl.BlockSpec((BLK, 128), lambda k: (k, 0)),
            pl.BlockSpec((BLK, 128), lambda k: (k, 0)),
        ],
        out_shape=[
            jax.ShapeDtypeStruct((N_PAD, 128), f32),
            jax.ShapeDtypeStruct((N_PAD, 128), f32),
        ],
    )(scene_pos, gw_pad, _PE_G_PAD, _ONEHOT_PAD, ws_pad, wed_pad,
      wg_pad, bg_pad, tbl_pad)


def _sc_body(zeros_h, esrc0, esrc1, edst, c2, acc0_o, acc1_o,
             acc_sh, sidxb, didxb, rows0, rows1,
             sem0, sem1, ssem0, ssem1):
    cid = lax.axis_index("c")
    sid = lax.axis_index("s")
    rb = sid * RPT

    pltpu.sync_copy(zeros_h, rows0)
    for z in range(RPT // CH):
        pltpu.sync_copy(rows0, acc_sh.at[pl.ds(rb + z * CH, CH)])
    rem = RPT % CH
    if rem:
        pltpu.sync_copy(rows0.at[pl.ds(0, rem)],
                        acc_sh.at[pl.ds(rb + (RPT // CH) * CH, rem)])

    plsc.subcore_barrier()

    rows = (rows0, rows1)
    sems = (sem0, sem1)
    ssems = (ssem0, ssem1)

    def make_batch(esrc):
        def batch(b, carry):
            base = sid * ER_PER_TILE + b * BATCH_R
            pltpu.sync_copy(esrc.at[pl.ds(base, BATCH_R)], sidxb)
            pltpu.sync_copy(edst.at[pl.ds(base, BATCH_R)], didxb)
            gd = {}
            sd = {}
            for t in range(NRING - 1):
                gd[t] = pltpu.async_copy(c2.at[sidxb.at[t]],
                                         rows[t], sems[t])
            for t in range(BATCH_R):
                slot = t % NRING
                nt = t + NRING - 1
                if nt < BATCH_R:
                    sprev = sd.get(t - 1)
                    if sprev is not None:
                        sprev.wait()
                    gd[nt] = pltpu.async_copy(c2.at[sidxb.at[nt]],
                                              rows[nt % NRING], sems[nt % NRING])
                gd[t].wait()
                sd[t] = pltpu.async_copy(rows[slot], acc_sh.at[didxb.at[t]],
                                         ssems[slot], add=True)
            for t in range(BATCH_R - NRING, BATCH_R):
                sd[t].wait()
            return carry
        return batch

    @pl.when(cid == 0)
    def _edges0():
        lax.fori_loop(0, NBATCH, make_batch(esrc0), 0)

    @pl.when(cid == 1)
    def _edges1():
        lax.fori_loop(0, NBATCH, make_batch(esrc1), 0)

    plsc.subcore_barrier()

    @pl.when(cid == 0)
    def _out0():
        pltpu.sync_copy(acc_sh.at[pl.ds(rb, RPT)], acc0_o.at[pl.ds(rb, RPT)])

    @pl.when(cid == 1)
    def _out1():
        pltpu.sync_copy(acc_sh.at[pl.ds(rb, RPT)], acc1_o.at[pl.ds(rb, RPT)])


def _edge_scatter(zeros_h, esrc0, esrc1, edst, c2):
    f32 = jnp.float32
    i32 = jnp.int32
    mesh = plsc.VectorSubcoreMesh(core_axis_name="c", subcore_axis_name="s",
                                  num_cores=NCORE, num_subcores=NSUB)
    fn = pl.kernel(
        _sc_body,
        out_type=[
            jax.ShapeDtypeStruct((N_SC, 64), f32),
            jax.ShapeDtypeStruct((N_SC, 64), f32),
        ],
        mesh=mesh,
        scratch_types=[
            pltpu.VMEM_SHARED((N_SC, 64), f32),
            pltpu.VMEM((BATCH_R, CH), i32),
            pltpu.VMEM((BATCH_R, CH), i32),
            pltpu.VMEM((CH, 64), f32),
            pltpu.VMEM((CH, 64), f32),
            pltpu.SemaphoreType.DMA,
            pltpu.SemaphoreType.DMA,
            pltpu.SemaphoreType.DMA,
            pltpu.SemaphoreType.DMA,
        ],
        compiler_params=pltpu.CompilerParams(needs_layout_passes=False,
                                             use_tc_tiling_on_sc=False),
    )
    return fn(zeros_h, esrc0, esrc1, edst, c2)


DEG_BLK = 4096


def _deg_body(drow_ref, o_ref):
    k = pl.program_id(0)
    d = drow_ref[...].reshape(1, DEG_BLK)
    hi_row = lax.shift_right_logical(d, 7)
    lo_row = lax.bitwise_and(d, 127)
    a = (lax.broadcasted_iota(jnp.int32, (DEG_HI, DEG_BLK), 0)
         == hi_row).astype(jnp.bfloat16)
    bt = (lax.broadcasted_iota(jnp.int32, (128, DEG_BLK), 0)
          == lo_row).astype(jnp.bfloat16)
    contrib = lax.dot_general(a, bt, (((1,), (1,)), ((), ())),
                              preferred_element_type=jnp.float32)

    @pl.when(k == 0)
    def _init():
        o_ref[...] = contrib

    @pl.when(k > 0)
    def _acc():
        o_ref[...] += contrib


def _degree(edst_row):
    return pl.pallas_call(
        _deg_body,
        grid=(E // DEG_BLK,),
        in_specs=[
            pl.BlockSpec((1, 1, DEG_BLK), lambda k: (k, 0, 0)),
        ],
        out_specs=pl.BlockSpec((DEG_HI, 128), lambda k: (0, 0)),
        out_shape=jax.ShapeDtypeStruct((DEG_HI, 128), jnp.float32),
    )(edst_row)


def _comb_body(a0_ref, a1_ref, xf_ref, bd_ref, dg_ref, o_ref):
    o_ref[...] = (jnp.concatenate([a0_ref[...], a1_ref[...]], axis=1)
                  + xf_ref[...] + dg_ref[...] * bd_ref[...])


def _combine(acc0, acc1, xf, bd, deg_col):
    return pl.pallas_call(
        _comb_body,
        grid=(NSUB,),
        in_specs=[
            pl.BlockSpec((RPT, 64), lambda k: (k, 0)),
            pl.BlockSpec((RPT, 64), lambda k: (k, 0)),
            pl.BlockSpec((RPT, 128), lambda k: (k, 0)),
            pl.BlockSpec((RPT, 128), lambda k: (k, 0)),
            pl.BlockSpec((RPT, 1), lambda k: (k, 0)),
        ],
        out_specs=pl.BlockSpec((RPT, 128), lambda k: (k, 0)),
        out_shape=jax.ShapeDtypeStruct((N, 128), jnp.float32),
    )(acc0, acc1, xf, bd, deg_col)


def kernel(scene_pos, gripper_width, edge_index, W_scene, W_grip, b_grip,
           gripper_table, W_edge):
    f32 = jnp.float32
    ws_pad = jnp.zeros((64, 128), f32).at[:DPOS].set(W_scene.astype(f32))
    wes_pad = jnp.zeros((64, 128), f32).at[:DPOS].set(W_edge[:DPOS].astype(f32))
    wed_pad = jnp.zeros((64, 128), f32).at[:DPOS].set(W_edge[DPOS:2 * DPOS].astype(f32))
    wg_pad = jnp.zeros((8, 64), f32).at[0:1].set(W_grip.astype(f32))
    bg_pad = jnp.zeros((8, 64), f32).at[0].set(b_grip.astype(f32))
    tbl_pad = jnp.zeros((64, 64), f32).at[:NG * (P + 1)].set(gripper_table.astype(f32))
    gw_pad = jnp.zeros((G_PAD, 1), f32).at[:NGN].set(gripper_width.astype(f32))

    sp = scene_pos.astype(f32)
    cc = _featA(sp, gw_pad, ws_pad + wes_pad, wes_pad, wg_pad, bg_pad, tbl_pad)
    xf, bd = _featB(sp, gw_pad, ws_pad, wed_pad, wg_pad, bg_pad, tbl_pad)

    ei = edge_index.astype(jnp.int32)
    esrc = ei[0]
    esrc0 = (esrc * 2).reshape(EROWS, CH)
    esrc1 = (esrc * 2 + 1).reshape(EROWS, CH)
    edst = ei[1].reshape(EROWS, CH)
    c2 = cc.reshape(2 * N_PAD, 64)
    zeros_h = jnp.zeros((CH, 64), f32)
    acc0, acc1 = _edge_scatter(zeros_h, esrc0, esrc1, edst, c2)

    deg = _degree(ei[1].reshape(E // DEG_BLK, 1, DEG_BLK))
    deg_col = deg.reshape(-1)[:N_SC].reshape(N_SC, 1)
    return _combine(acc0, acc1, xf, bd, deg_col)


# speedup vs baseline: 1.1925x; 1.1925x over previous
import functools

import jax
import jax.numpy as jnp
import numpy as np
from jax import lax
from jax.experimental import pallas as pl
from jax.experimental.pallas import tpu as pltpu
from jax.experimental.pallas import tpu_sc as plsc

B, D, T, S, NG, P = 16, 2, 8, 64, 6, 8
NS = B * (D * T * S + S + P * S)
NGN = B * (D * T * NG + NG + P * NG)
N = NS + NGN
E = 262144
EMBD = 128
GSTATE = 64
NFREQ = 10
DPOS = 3 * (1 + 2 * NFREQ)

BLK = 1024
N_PAD = 28672
NBLK = N_PAD // BLK
NSBLK = NS // BLK
G_PAD = N_PAD - NS

NCORE, NSUB = 2, 16
N_SC = 28032
RPT = N_SC // NSUB
CH = 128
EROWS = E // CH
ER_PER_TILE = EROWS // NSUB
BATCH_R = 8
NBATCH = ER_PER_TILE // BATCH_R
NRING = 2
DEG_HI = 224


def _np_pos_enc(x):
    freqs = 2.0 ** np.arange(NFREQ)
    xf = x[:, :, None] * freqs
    enc = np.concatenate([np.sin(xf), np.cos(xf)], axis=-1)
    enc = enc.reshape(x.shape[0], 3 * 2 * NFREQ)
    return np.concatenate([x, enc], axis=-1).astype(np.float32)


def _static_gripper():
    node1 = np.broadcast_to(np.arange(NG)[None, None, None, :], (B, D, T, NG)).reshape(-1)
    node2 = np.broadcast_to(np.arange(NG)[None, :], (B, NG)).reshape(-1)
    node3 = np.broadcast_to(np.arange(NG)[None, None, :], (B, P, NG)).reshape(-1)
    t_act = np.broadcast_to(np.arange(P)[None, :, None], (B, P, NG)).reshape(-1)
    gripper_node = np.concatenate([node1, node2, node3])
    embd_idx = np.concatenate([node1, node2, node3 + NG * t_act])
    gnp_pos = np.array([[0.0, 0.0, 0.0], [0.0, 0.0, -0.03], [0.0, 0.03, 0.0],
                        [0.0, -0.03, 0.0], [0.0, 0.03, 0.03], [0.0, -0.03, 0.03]],
                       dtype=np.float32) * 2.0
    pe_g = _np_pos_enc(gnp_pos[gripper_node])
    pe_g_pad = np.zeros((G_PAD, 64), np.float32)
    pe_g_pad[:NGN, :DPOS] = pe_g
    onehot = np.zeros((G_PAD, 64), np.float32)
    onehot[np.arange(NGN), embd_idx] = 1.0
    return pe_g_pad, onehot


_PE_G_PAD, _ONEHOT_PAD = _static_gripper()
_FREQS = (2.0 ** np.arange(NFREQ, dtype=np.float32)).reshape(1, NFREQ)


def _pe_scene(pos):
    colb = lax.broadcasted_iota(jnp.int32, (BLK, 64), 1)
    cm3 = colb - 3
    istrig = (colb >= 3) & (colb < 63)
    iscos = istrig & (lax.rem(cm3, 20) >= 10)
    coord = jnp.clip(jnp.where(colb < 3, colb, cm3 // 20), 0, 2)
    k10 = lax.rem(lax.rem(cm3 + 20, 20), 10)
    freq = jnp.exp2(k10.astype(jnp.float32))
    val = jnp.where(coord == 0, pos[:, 0:1],
                    jnp.where(coord == 1, pos[:, 1:2], pos[:, 2:3]))
    ang = val * jnp.where(istrig, freq, jnp.float32(1.0))
    shifted = ang + jnp.where(iscos, jnp.float32(np.pi / 2), jnp.float32(0))
    pe = jnp.where(istrig, jnp.sin(shifted), ang)
    return jnp.where(colb >= 63, jnp.float32(0.0), pe)


def _gemb_gst(oh_ref, tbl_ref, gw_ref, wg_ref, bg_ref):
    gemb = jnp.dot(oh_ref[...], tbl_ref[...], preferred_element_type=jnp.float32)
    gst = gw_ref[...] * wg_ref[0:1, :] + bg_ref[0:1, :]
    return jnp.concatenate([gemb, gst], axis=1)


def _feat_body(pos_ref, gw_ref, peg_ref, oh_ref, ws_ref, wes_ref, wed_ref,
               wg_ref, bg_ref, tbl_ref, xf_ref, cc_ref, bd_ref):
    k = pl.program_id(0)

    @pl.when(k < NSBLK)
    def _scene():
        pe = _pe_scene(pos_ref[...])
        x = jnp.dot(pe, ws_ref[...], preferred_element_type=jnp.float32)
        a = jnp.dot(pe, wes_ref[...], preferred_element_type=jnp.float32)
        xf_ref[...] = x
        cc_ref[...] = x + a
        bd_ref[...] = jnp.dot(pe, wed_ref[...], preferred_element_type=jnp.float32)

    @pl.when(k >= NSBLK)
    def _grip():
        peg = peg_ref[...]
        x = _gemb_gst(oh_ref, tbl_ref, gw_ref, wg_ref, bg_ref)
        a = jnp.dot(peg, wes_ref[...], preferred_element_type=jnp.float32)
        xf_ref[...] = x
        cc_ref[...] = x + a
        bd_ref[...] = jnp.dot(peg, wed_ref[...], preferred_element_type=jnp.float32)


_SCENE_IX = lambda k: (jnp.minimum(k, NSBLK - 1), 0)
_GRIP_IX = lambda k: (jnp.maximum(k - NSBLK, 0), 0)
_FULL_IX = lambda k: (0, 0)
_COMMON_SPECS = [
    pl.BlockSpec((BLK, 3), _SCENE_IX),
    pl.BlockSpec((BLK, 1), _GRIP_IX),
    pl.BlockSpec((BLK, 64), _GRIP_IX),
    pl.BlockSpec((BLK, 64), _GRIP_IX),
]
_W_SPECS = [
    pl.BlockSpec((8, 64), _FULL_IX),
    pl.BlockSpec((8, 64), _FULL_IX),
    pl.BlockSpec((64, 64), _FULL_IX),
]


def _features(scene_pos, gw_pad, ws_pad, wes_pad, wed_pad, wg_pad, bg_pad, tbl_pad):
    f32 = jnp.float32
    return pl.pallas_call(
        _feat_body,
        grid=(NBLK,),
        in_specs=(_COMMON_SPECS
                  + [pl.BlockSpec((64, 128), _FULL_IX),
                     pl.BlockSpec((64, 128), _FULL_IX),
                     pl.BlockSpec((64, 128), _FULL_IX)] + _W_SPECS),
        out_specs=[
            pl.BlockSpec((BLK, 128), lambda k: (k, 0)),
            pl.BlockSpec((BLK, 128), lambda k: (k, 0)),
            pl.BlockSpec((BLK, 128), lambda k: (k, 0)),
        ],
        out_shape=[
            jax.ShapeDtypeStruct((N_PAD, 128), f32),
            jax.ShapeDtypeStruct((N_PAD, 128), f32),
            jax.ShapeDtypeStruct((N_PAD, 128), f32),
        ],
    )(scene_pos, gw_pad, _PE_G_PAD, _ONEHOT_PAD, ws_pad, wes_pad, wed_pad,
      wg_pad, bg_pad, tbl_pad)


def _sc_body(zeros_h, esrc0, esrc1, edst, c2, acc0_o, acc1_o,
             acc_sh, sidxb, didxb, rows0, rows1,
             sem0, sem1, ssem0, ssem1):
    cid = lax.axis_index("c")
    sid = lax.axis_index("s")
    rb = sid * RPT

    pltpu.sync_copy(zeros_h, rows0)
    for z in range(RPT // CH):
        pltpu.sync_copy(rows0, acc_sh.at[pl.ds(rb + z * CH, CH)])
    rem = RPT % CH
    if rem:
        pltpu.sync_copy(rows0.at[pl.ds(0, rem)],
                        acc_sh.at[pl.ds(rb + (RPT // CH) * CH, rem)])

    plsc.subcore_barrier()

    rows = (rows0, rows1)
    sems = (sem0, sem1)
    ssems = (ssem0, ssem1)

    def make_batch(esrc):
        def batch(b, carry):
            base = sid * ER_PER_TILE + b * BATCH_R
            pltpu.sync_copy(esrc.at[pl.ds(base, BATCH_R)], sidxb)
            pltpu.sync_copy(edst.at[pl.ds(base, BATCH_R)], didxb)
            gd = {}
            sd = {}
            for t in range(NRING - 1):
                gd[t] = pltpu.async_copy(c2.at[sidxb.at[t]],
                                         rows[t], sems[t])
            for t in range(BATCH_R):
                slot = t % NRING
                nt = t + NRING - 1
                if nt < BATCH_R:
                    sprev = sd.get(t - 1)
                    if sprev is not None:
                        sprev.wait()
                    gd[nt] = pltpu.async_copy(c2.at[sidxb.at[nt]],
                                              rows[nt % NRING], sems[nt % NRING])
                gd[t].wait()
                sd[t] = pltpu.async_copy(rows[slot], acc_sh.at[didxb.at[t]],
                                         ssems[slot], add=True)
            for t in range(BATCH_R - NRING, BATCH_R):
                sd[t].wait()
            return carry
        return batch

    @pl.when(cid == 0)
    def _edges0():
        lax.fori_loop(0, NBATCH, make_batch(esrc0), 0)

    @pl.when(cid == 1)
    def _edges1():
        lax.fori_loop(0, NBATCH, make_batch(esrc1), 0)

    plsc.subcore_barrier()

    @pl.when(cid == 0)
    def _out0():
        pltpu.sync_copy(acc_sh.at[pl.ds(rb, RPT)], acc0_o.at[pl.ds(rb, RPT)])

    @pl.when(cid == 1)
    def _out1():
        pltpu.sync_copy(acc_sh.at[pl.ds(rb, RPT)], acc1_o.at[pl.ds(rb, RPT)])


def _edge_scatter(zeros_h, esrc0, esrc1, edst, c2):
    f32 = jnp.float32
    i32 = jnp.int32
    mesh = plsc.VectorSubcoreMesh(core_axis_name="c", subcore_axis_name="s",
                                  num_cores=NCORE, num_subcores=NSUB)
    fn = pl.kernel(
        _sc_body,
        out_type=[
            jax.ShapeDtypeStruct((N_SC, 64), f32),
            jax.ShapeDtypeStruct((N_SC, 64), f32),
        ],
        mesh=mesh,
        scratch_types=[
            pltpu.VMEM_SHARED((N_SC, 64), f32),
            pltpu.VMEM((BATCH_R, CH), i32),
            pltpu.VMEM((BATCH_R, CH), i32),
            pltpu.VMEM((CH, 64), f32),
            pltpu.VMEM((CH, 64), f32),
            pltpu.SemaphoreType.DMA,
            pltpu.SemaphoreType.DMA,
            pltpu.SemaphoreType.DMA,
            pltpu.SemaphoreType.DMA,
        ],
        compiler_params=pltpu.CompilerParams(needs_layout_passes=False,
                                             use_tc_tiling_on_sc=False),
    )
    return fn(zeros_h, esrc0, esrc1, edst, c2)


DEG_BLK = 4096


def _deg_body(drow_ref, o_ref):
    k = pl.program_id(0)
    d = drow_ref[...].reshape(1, DEG_BLK)
    hi_row = lax.shift_right_logical(d, 7)
    lo_row = lax.bitwise_and(d, 127)
    a = (lax.broadcasted_iota(jnp.int32, (DEG_HI, DEG_BLK), 0)
         == hi_row).astype(jnp.bfloat16)
    bt = (lax.broadcasted_iota(jnp.int32, (128, DEG_BLK), 0)
          == lo_row).astype(jnp.bfloat16)
    contrib = lax.dot_general(a, bt, (((1,), (1,)), ((), ())),
                              preferred_element_type=jnp.float32)

    @pl.when(k == 0)
    def _init():
        o_ref[...] = contrib

    @pl.when(k > 0)
    def _acc():
        o_ref[...] += contrib


def _degree(edst_row):
    return pl.pallas_call(
        _deg_body,
        grid=(E // DEG_BLK,),
        in_specs=[
            pl.BlockSpec((1, 1, DEG_BLK), lambda k: (k, 0, 0)),
        ],
        out_specs=pl.BlockSpec((DEG_HI, 128), lambda k: (0, 0)),
        out_shape=jax.ShapeDtypeStruct((DEG_HI, 128), jnp.float32),
    )(edst_row)


def _comb_body(a0_ref, a1_ref, xf_ref, bd_ref, dg_ref, o_ref):
    o_ref[...] = (jnp.concatenate([a0_ref[...], a1_ref[...]], axis=1)
                  + xf_ref[...] + dg_ref[...] * bd_ref[...])


def _combine(acc0, acc1, xf, bd, deg_col):
    return pl.pallas_call(
        _comb_body,
        grid=(NSUB,),
        in_specs=[
            pl.BlockSpec((RPT, 64), lambda k: (k, 0)),
            pl.BlockSpec((RPT, 64), lambda k: (k, 0)),
            pl.BlockSpec((RPT, 128), lambda k: (k, 0)),
            pl.BlockSpec((RPT, 128), lambda k: (k, 0)),
            pl.BlockSpec((RPT, 1), lambda k: (k, 0)),
        ],
        out_specs=pl.BlockSpec((RPT, 128), lambda k: (k, 0)),
        out_shape=jax.ShapeDtypeStruct((N, 128), jnp.float32),
    )(acc0, acc1, xf, bd, deg_col)


def kernel(scene_pos, gripper_width, edge_index, W_scene, W_grip, b_grip,
           gripper_table, W_edge):
    f32 = jnp.float32
    ws_pad = jnp.zeros((64, 128), f32).at[:DPOS].set(W_scene.astype(f32))
    wes_pad = jnp.zeros((64, 128), f32).at[:DPOS].set(W_edge[:DPOS].astype(f32))
    wed_pad = jnp.zeros((64, 128), f32).at[:DPOS].set(W_edge[DPOS:2 * DPOS].astype(f32))
    wg_pad = jnp.zeros((8, 64), f32).at[0:1].set(W_grip.astype(f32))
    bg_pad = jnp.zeros((8, 64), f32).at[0].set(b_grip.astype(f32))
    tbl_pad = jnp.zeros((64, 64), f32).at[:NG * (P + 1)].set(gripper_table.astype(f32))
    gw_pad = jnp.zeros((G_PAD, 1), f32).at[:NGN].set(gripper_width.astype(f32))

    xf, cc, bd = _features(scene_pos.astype(f32), gw_pad, ws_pad, wes_pad,
                           wed_pad, wg_pad, bg_pad, tbl_pad)

    ei = edge_index.astype(jnp.int32)
    esrc = ei[0]
    esrc0 = (esrc * 2).reshape(EROWS, CH)
    esrc1 = (esrc * 2 + 1).reshape(EROWS, CH)
    edst = ei[1].reshape(EROWS, CH)
    c2 = cc.reshape(2 * N_PAD, 64)
    zeros_h = jnp.zeros((CH, 64), f32)
    acc0, acc1 = _edge_scatter(zeros_h, esrc0, esrc1, edst, c2)

    deg = _degree(ei[1].reshape(E // DEG_BLK, 1, DEG_BLK))
    deg_col = deg.reshape(-1)[:N_SC].reshape(N_SC, 1)
    return _combine(acc0, acc1, xf, bd, deg_col)
